# Initial kernel scaffold; baseline (speedup 1.0000x reference)
import jax
import jax.numpy as jnp
from jax.experimental import pallas as pl


def kernel(embedding_a, embedding_b, b_y):
    raise NotImplementedError("write your pallas kernel here")


# speedup vs baseline: 3.6272x; 3.6272x over previous
import functools

import jax
import jax.numpy as jnp
from jax import lax
from jax.experimental import pallas as pl
from jax.experimental.pallas import tpu as pltpu
from jax.experimental.pallas import tpu_sc as plsc

Q = 1024
D = 64
K = 100000
CH = 128
KC = 2048
KP = 100352
NCHUNK = KP // CH
NSTEP = KP // KC
NCLS = 32
TOPK = 10
EPS = 1e-8
NEG = -3.0
NEGSEL = -3.4e38
IMAX = 2**31 - 1


def _phase_a_body(a_ref, b_ref, sim_ref, mt_ref):
    j = pl.program_id(0)
    a = a_ref[...]
    an = jnp.sqrt(jnp.sum(a * a, axis=1, keepdims=True))
    a = a / jnp.maximum(an, EPS)
    b = b_ref[...]
    bn = jnp.sqrt(jnp.sum(b * b, axis=1, keepdims=True))
    b = b / jnp.maximum(bn, EPS)
    sim = lax.dot_general(a, b, (((1,), (1,)), ((), ())),
                          preferred_element_type=jnp.float32)
    kcol = j * KC + lax.broadcasted_iota(jnp.int32, (Q, KC), 1)
    sim = jnp.where(kcol < K, sim, NEG)
    sim_ref[...] = sim
    rows = []
    for t in range(KC // CH):
        sub = sim[:, t * CH:(t + 1) * CH]
        rows.append(jnp.max(sub, axis=1)[None, :])
    mt_ref[...] = jnp.concatenate(rows, axis=0)


def _phase_b_body(mt_ref, cid_ref, thr_ref):
    work = mt_ref[...]
    row_iota = lax.broadcasted_iota(jnp.int32, (NCHUNK, Q), 0)
    mv = None
    for s in range(TOPK):
        mv = jnp.max(work, axis=0)
        cid = jnp.min(jnp.where(work == mv[None, :], row_iota, IMAX), axis=0)
        cid_ref[pl.ds(s, 1), :] = cid[None, :]
        work = jnp.where(row_iota == cid[None, :], -jnp.inf, work)
    cid_ref[pl.ds(TOPK, 16 - TOPK), :] = jnp.zeros((16 - TOPK, Q), jnp.int32)
    thr_ref[...] = jnp.broadcast_to(mv[None, :], (8, Q))


def _phase_c_body(sim_hbm, cid_hbm, by_hbm, out_hbm,
                  by_v, cid_v, gbuf, obuf,
                  sem0, sem1):
    nc = 2
    wid = lax.axis_index("s") * nc + lax.axis_index("c")
    qpw = Q // 32
    q0 = wid * qpw
    pltpu.sync_copy(by_hbm, by_v)
    pltpu.sync_copy(cid_hbm.at[pl.ds(q0 * 16, qpw * 16)], cid_v)
    iota = lax.iota(jnp.int32, 16)

    def qbody(ql, carry):
        qg = q0 + ql
        cidvec = cid_v[pl.ds(ql * 16, 16)]
        gidx = cidvec + qg * NCHUNK
        pltpu.async_copy(sim_hbm.at[gidx], gbuf, sem0).wait()

        def sel_body(s, carry):
            selv, seli, pv, pi = carry
            bestv = jnp.full((16,), NEGSEL, jnp.float32)
            besti = jnp.full((16,), IMAX, jnp.int32)
            for jj in range(TOPK):
                base = cidvec[jj] * CH
                for t in range(CH // 16):
                    v = gbuf[jj, pl.ds(t * 16, 16)]
                    gi = iota + (base + t * 16)
                    elig = (v < pv) | ((v == pv) & (gi > pi))
                    better = elig & ((v > bestv) | ((v == bestv) & (gi < besti)))
                    bestv = jnp.where(better, v, bestv)
                    besti = jnp.where(better, gi, besti)
            mv = bestv
            for sh in (8, 4, 2, 1):
                perm = lax.bitwise_xor(iota, sh)
                mv = jnp.maximum(mv, jnp.take(mv, perm))
            mi = jnp.where(bestv == mv, besti, IMAX)
            for sh in (8, 4, 2, 1):
                perm = lax.bitwise_xor(iota, sh)
                mi = jnp.minimum(mi, jnp.take(mi, perm))
            sel_here = iota == s
            selv = jnp.where(sel_here, mv, selv)
            seli = jnp.where(sel_here, mi, seli)
            return (selv, seli, mv, mi)

        zero16f = jnp.zeros((16,), jnp.float32)
        zero16i = jnp.zeros((16,), jnp.int32)
        selv, seli, _, _ = lax.fori_loop(
            0, TOPK, sel_body,
            (zero16f, zero16i,
             jnp.full((16,), 3.4e38, jnp.float32),
             jnp.full((16,), -1, jnp.int32)))
        w = jnp.exp(selv)
        lbl = plsc.load_gather(by_v, [seli])
        acc0 = jnp.zeros((16,), jnp.float32)
        acc1 = jnp.zeros((16,), jnp.float32)
        for sl in range(TOPK):
            l = lbl[sl]
            ws = w[sl]
            acc0 = acc0 + jnp.where(iota == l, ws, 0.0)
            acc1 = acc1 + jnp.where(iota == (l - 16), ws, 0.0)
        obuf[0, pl.ds(0, 16)] = acc0
        obuf[0, pl.ds(16, 16)] = acc1
        pltpu.sync_copy(obuf, out_hbm.at[pl.ds(qg, 1)])
        return carry

    lax.fori_loop(0, qpw, qbody, 0)


def kernel(embedding_a, embedding_b, b_y):
    b_pad = jnp.pad(embedding_b, ((0, KP - K), (0, 0)))
    by_pad = jnp.pad(b_y, (0, KP - K)).astype(jnp.int32)

    sim, mt = pl.pallas_call(
        _phase_a_body,
        grid=(NSTEP,),
        in_specs=[
            pl.BlockSpec((Q, D), lambda j: (0, 0)),
            pl.BlockSpec((KC, D), lambda j: (j, 0)),
        ],
        out_specs=[
            pl.BlockSpec((Q, KC), lambda j: (0, j)),
            pl.BlockSpec((KC // CH, Q), lambda j: (j, 0)),
        ],
        out_shape=[
            jax.ShapeDtypeStruct((Q, KP), jnp.float32),
            jax.ShapeDtypeStruct((NCHUNK, Q), jnp.float32),
        ],
    )(embedding_a, b_pad)

    cid16, _thr = pl.pallas_call(
        _phase_b_body,
        out_shape=[
            jax.ShapeDtypeStruct((16, Q), jnp.int32),
            jax.ShapeDtypeStruct((8, Q), jnp.float32),
        ],
    )(mt)

    cid_flat = cid16.T.reshape(Q * 16)
    sim_rows = sim.reshape(Q * NCHUNK, CH)

    sc = pl.kernel(
        _phase_c_body,
        out_type=jax.ShapeDtypeStruct((Q, NCLS), jnp.float32),
        mesh=plsc.VectorSubcoreMesh(core_axis_name="c", subcore_axis_name="s"),
        compiler_params=pltpu.CompilerParams(needs_layout_passes=False),
        scratch_types=[
            pltpu.VMEM((KP,), jnp.int32),
            pltpu.VMEM((Q // 32 * 16,), jnp.int32),
            pltpu.VMEM((16, CH), jnp.float32),
            pltpu.VMEM((1, NCLS), jnp.float32),
            pltpu.SemaphoreType.DMA,
            pltpu.SemaphoreType.DMA,
        ],
    )
    return sc(sim_rows, cid_flat, by_pad)


# speedup vs baseline: 4.5776x; 1.2620x over previous
import functools

import jax
import jax.numpy as jnp
from jax import lax
from jax.experimental import pallas as pl
from jax.experimental.pallas import tpu as pltpu
from jax.experimental.pallas import tpu_sc as plsc

Q = 1024
D = 64
K = 100000
CH = 128
KC = 2048
KP = 100352
NCHUNK = KP // CH
NSTEP = KP // KC
NCLS = 32
TOPK = 10
EPS = 1e-8
NEG = -3.0
NEGSEL = -3.4e38
IMAX = 2**31 - 1


def _phase_a_body(a_ref, b_ref, sim_ref, mt_ref):
    j = pl.program_id(0)
    a = a_ref[...]
    an = jnp.sqrt(jnp.sum(a * a, axis=1, keepdims=True))
    a = a / jnp.maximum(an, EPS)
    b = b_ref[...]
    bn = jnp.sqrt(jnp.sum(b * b, axis=1, keepdims=True))
    b = b / jnp.maximum(bn, EPS)
    sim = lax.dot_general(a, b, (((1,), (1,)), ((), ())),
                          preferred_element_type=jnp.float32)
    kcol = j * KC + lax.broadcasted_iota(jnp.int32, (Q, KC), 1)
    sim = jnp.where(kcol < K, sim, NEG)
    sim_ref[...] = sim
    rows = []
    for t in range(KC // CH):
        sub = sim[:, t * CH:(t + 1) * CH]
        rows.append(jnp.max(sub, axis=1)[None, :])
    mt_ref[...] = jnp.concatenate(rows, axis=0)


def _phase_b_body(mt_ref, cid_ref, thr_ref):
    work = mt_ref[...]
    row_iota = lax.broadcasted_iota(jnp.int32, (NCHUNK, Q), 0)
    mv = None
    cids = []
    for s in range(TOPK):
        mv = jnp.max(work, axis=0)
        cid = jnp.min(jnp.where(work == mv[None, :], row_iota, IMAX), axis=0)
        cids.append(cid)
        work = jnp.where(row_iota == cid[None, :], -jnp.inf, work)
    for a in range(TOPK - 1):
        for b in range(TOPK - 1 - a):
            lo = jnp.minimum(cids[b], cids[b + 1])
            hi = jnp.maximum(cids[b], cids[b + 1])
            cids[b], cids[b + 1] = lo, hi
    for s in range(TOPK):
        cid_ref[pl.ds(s, 1), :] = cids[s][None, :]
    cid_ref[pl.ds(TOPK, 16 - TOPK), :] = jnp.zeros((16 - TOPK, Q), jnp.int32)
    thr_ref[...] = jnp.broadcast_to(mv[None, :], (8, Q))


def _phase_c_body(sim_hbm, cid_hbm, thr_hbm, by_hbm, out_hbm,
                  by_v, cid_v, thr_v, gbufa, gbufb, svv, siv, obuf,
                  sema, semb):
    nc = 2
    wid = lax.axis_index("s") * nc + lax.axis_index("c")
    qpw = Q // 32
    q0 = wid * qpw
    pltpu.sync_copy(by_hbm, by_v)
    pltpu.sync_copy(cid_hbm.at[pl.ds(q0 * 16, qpw * 16)], cid_v)
    pltpu.sync_copy(thr_hbm.at[pl.ds(q0, qpw)], thr_v.at[pl.ds(0, qpw)])
    iota = lax.iota(jnp.int32, 16)

    def _gather(ql, buf, sem):
        cidvec = cid_v[pl.ds(ql * 16, 16)]
        gidx = cidvec + (q0 + ql) * NCHUNK
        return pltpu.make_async_copy(sim_hbm.at[gidx], buf, sem)

    def _process(ql, buf):
        thrs = thr_v[pl.ds(ql, 16)][0]
        off = jnp.int32(0)
        for jj in range(TOPK):
            for t in range(CH // 16):
                v = buf[jj, pl.ds(t * 16, 16)]
                keep = v >= thrs
                p = iota + (jj * CH + t * 16)
                plsc.store_compressed(svv.at[pl.ds(off, 16)], v, mask=keep)
                plsc.store_compressed(siv.at[pl.ds(off, 16)], p, mask=keep)
                cnt = plsc.all_reduce_population_count(keep)
                cnt = cnt[0] if jnp.ndim(cnt) else cnt
                off = off + cnt
        svv[pl.ds(off, 16)] = jnp.full((16,), NEGSEL, jnp.float32)
        siv[pl.ds(off, 16)] = jnp.full((16,), IMAX, jnp.int32)
        nv = (off + 15) // 16

        def sel_body(s, carry):
            selv, seli, pv, pi = carry

            def scan_body(k, c2):
                bestv, besti = c2
                sv = svv[pl.ds(k * 16, 16)]
                si = siv[pl.ds(k * 16, 16)]
                elig = (sv < pv) | ((sv == pv) & (si > pi))
                better = elig & ((sv > bestv) | ((sv == bestv) & (si < besti)))
                return (jnp.where(better, sv, bestv),
                        jnp.where(better, si, besti))

            bestv, besti = lax.fori_loop(
                0, nv, scan_body,
                (jnp.full((16,), NEGSEL, jnp.float32),
                 jnp.full((16,), IMAX, jnp.int32)))
            mv = bestv
            for sh in (8, 4, 2, 1):
                mv = jnp.maximum(mv, jnp.take(mv, lax.bitwise_xor(iota, sh)))
            mi = jnp.where(bestv == mv, besti, IMAX)
            for sh in (8, 4, 2, 1):
                mi = jnp.minimum(mi, jnp.take(mi, lax.bitwise_xor(iota, sh)))
            sel_here = iota == s
            return (jnp.where(sel_here, mv, selv),
                    jnp.where(sel_here, mi, seli),
                    mv, mi)

        selv, seli, _, _ = lax.fori_loop(
            0, TOPK, sel_body,
            (jnp.zeros((16,), jnp.float32), jnp.zeros((16,), jnp.int32),
             jnp.full((16,), 3.4e38, jnp.float32),
             jnp.full((16,), -1, jnp.int32)))
        jjv = seli // CH
        rem = seli - jjv * CH
        cidsel = plsc.load_gather(cid_v, [jjv + ql * 16])
        gsel = cidsel * CH + rem
        w = jnp.exp(selv)
        lbl = plsc.load_gather(by_v, [gsel])
        acc0 = jnp.zeros((16,), jnp.float32)
        acc1 = jnp.zeros((16,), jnp.float32)
        for sl in range(TOPK):
            l = lbl[sl]
            ws = w[sl]
            acc0 = acc0 + jnp.where(iota == l, ws, 0.0)
            acc1 = acc1 + jnp.where(iota == (l - 16), ws, 0.0)
        obuf[pl.ds(ql * NCLS, 16)] = acc0
        obuf[pl.ds(ql * NCLS + 16, 16)] = acc1

    _gather(0, gbufa, sema).start()
    _gather(1, gbufb, semb).start()

    def qbody(i, carry):
        ql0 = i * 2
        _gather(ql0, gbufa, sema).wait()
        _process(ql0, gbufa)

        @pl.when(ql0 + 2 < qpw)
        def _():
            _gather(ql0 + 2, gbufa, sema).start()

        _gather(ql0 + 1, gbufb, semb).wait()
        _process(ql0 + 1, gbufb)

        @pl.when(ql0 + 3 < qpw)
        def _():
            _gather(ql0 + 3, gbufb, semb).start()

        return carry

    lax.fori_loop(0, qpw // 2, qbody, 0)
    pltpu.sync_copy(obuf, out_hbm.at[pl.ds(q0 * NCLS, qpw * NCLS)])


def kernel(embedding_a, embedding_b, b_y):
    b_pad = jnp.pad(embedding_b, ((0, KP - K), (0, 0)))
    by_pad = jnp.pad(b_y, (0, KP - K)).astype(jnp.int32)

    sim, mt = pl.pallas_call(
        _phase_a_body,
        grid=(NSTEP,),
        in_specs=[
            pl.BlockSpec((Q, D), lambda j: (0, 0)),
            pl.BlockSpec((KC, D), lambda j: (j, 0)),
        ],
        out_specs=[
            pl.BlockSpec((Q, KC), lambda j: (0, j)),
            pl.BlockSpec((KC // CH, Q), lambda j: (j, 0)),
        ],
        out_shape=[
            jax.ShapeDtypeStruct((Q, KP), jnp.float32),
            jax.ShapeDtypeStruct((NCHUNK, Q), jnp.float32),
        ],
    )(embedding_a, b_pad)

    cid16, _thr = pl.pallas_call(
        _phase_b_body,
        out_shape=[
            jax.ShapeDtypeStruct((16, Q), jnp.int32),
            jax.ShapeDtypeStruct((8, Q), jnp.float32),
        ],
    )(mt)

    cid_flat = cid16.T.reshape(Q * 16)
    thr_q = _thr[0]
    sim_rows = sim.reshape(Q * NCHUNK, CH)

    sc = pl.kernel(
        _phase_c_body,
        out_type=jax.ShapeDtypeStruct((Q * NCLS,), jnp.float32),
        mesh=plsc.VectorSubcoreMesh(core_axis_name="c", subcore_axis_name="s"),
        compiler_params=pltpu.CompilerParams(needs_layout_passes=False),
        scratch_types=[
            pltpu.VMEM((KP,), jnp.int32),
            pltpu.VMEM((Q // 32 * 16,), jnp.int32),
            pltpu.VMEM((Q // 32 + 16,), jnp.float32),
            pltpu.VMEM((16, CH), jnp.float32),
            pltpu.VMEM((16, CH), jnp.float32),
            pltpu.VMEM((TOPK * CH + 32,), jnp.float32),
            pltpu.VMEM((TOPK * CH + 32,), jnp.int32),
            pltpu.VMEM((Q // 32 * NCLS,), jnp.float32),
            pltpu.SemaphoreType.DMA,
            pltpu.SemaphoreType.DMA,
        ],
    )
    return sc(sim_rows, cid_flat, thr_q, by_pad).reshape(Q, NCLS)


# speedup vs baseline: 4.7334x; 1.0340x over previous
import functools

import jax
import jax.numpy as jnp
from jax import lax
from jax.experimental import pallas as pl
from jax.experimental.pallas import tpu as pltpu
from jax.experimental.pallas import tpu_sc as plsc

Q = 1024
D = 64
K = 100000
CH = 128
KC = 2048
KP = 100352
NCHUNK = KP // CH
NSTEP = KP // KC
NCLS = 32
TOPK = 10
EPS = 1e-8
NEG = -3.0
NEGSEL = -3.4e38
IMAX = 2**31 - 1


def _phase_a_body(a_ref, b_ref, sim_ref, mt_ref):
    j = pl.program_id(0)
    a = a_ref[...]
    an = jnp.sqrt(jnp.sum(a * a, axis=1, keepdims=True))
    a = a / jnp.maximum(an, EPS)
    b = b_ref[...]
    bn = jnp.sqrt(jnp.sum(b * b, axis=1, keepdims=True))
    b = b / jnp.maximum(bn, EPS)
    sim = lax.dot_general(a, b, (((1,), (1,)), ((), ())),
                          preferred_element_type=jnp.float32)
    kcol = j * KC + lax.broadcasted_iota(jnp.int32, (Q, KC), 1)
    sim = jnp.where(kcol < K, sim, NEG)
    sim_ref[...] = sim
    rows = []
    for t in range(KC // CH):
        sub = sim[:, t * CH:(t + 1) * CH]
        rows.append(jnp.max(sub, axis=1)[None, :])
    mt_ref[...] = jnp.concatenate(rows, axis=0)


def _phase_b_body(mt_ref, cid_ref):
    work = mt_ref[...]
    row_iota = lax.broadcasted_iota(jnp.int32, (NCHUNK, Q), 0)
    mv = None
    cids = []
    for s in range(TOPK):
        mv = jnp.max(work, axis=0)
        cid = jnp.min(jnp.where(work == mv[None, :], row_iota, IMAX), axis=0)
        cids.append(cid)
        work = jnp.where(row_iota == cid[None, :], -jnp.inf, work)
    for a in range(TOPK - 1):
        for b in range(TOPK - 1 - a):
            lo = jnp.minimum(cids[b], cids[b + 1])
            hi = jnp.maximum(cids[b], cids[b + 1])
            cids[b], cids[b + 1] = lo, hi
    cols = [c[:, None] for c in cids]
    cols.append(lax.bitcast_convert_type(mv, jnp.int32)[:, None])
    cols.append(jnp.zeros((Q, 16 - TOPK - 1), jnp.int32))
    cid_ref[...] = jnp.concatenate(cols, axis=1)


def _phase_c_body(sim_hbm, cid_hbm, by_hbm, out_hbm,
                  by_v, cid_v, gbufa, gbufb, svv, siv, obuf,
                  sema, semb):
    nc = 2
    wid = lax.axis_index("s") * nc + lax.axis_index("c")
    qpw = Q // 32
    q0 = wid * qpw
    pltpu.sync_copy(by_hbm, by_v)
    pltpu.sync_copy(cid_hbm.at[pl.ds(q0 * 16, qpw * 16)], cid_v)
    iota = lax.iota(jnp.int32, 16)

    def _gather(ql, buf, sem):
        cidvec = cid_v[pl.ds(ql * 16, 16)]
        gidx = jnp.where(iota < TOPK, cidvec, 0) + (q0 + ql) * NCHUNK
        return pltpu.make_async_copy(sim_hbm.at[gidx], buf, sem)

    def _process(ql, buf):
        cidvec = cid_v[pl.ds(ql * 16, 16)]
        thrs = plsc.bitcast(cidvec, jnp.float32)[TOPK]
        off = jnp.int32(0)
        for jj in range(TOPK):
            for t in range(CH // 16):
                v = buf[jj, pl.ds(t * 16, 16)]
                keep = v >= thrs
                p = iota + (jj * CH + t * 16)
                plsc.store_compressed(svv.at[pl.ds(off, 16)], v, mask=keep)
                plsc.store_compressed(siv.at[pl.ds(off, 16)], p, mask=keep)
                cnt = plsc.all_reduce_population_count(keep)
                cnt = cnt[0] if jnp.ndim(cnt) else cnt
                off = off + cnt
        svv[pl.ds(off, 16)] = jnp.full((16,), NEGSEL, jnp.float32)
        siv[pl.ds(off, 16)] = jnp.full((16,), IMAX, jnp.int32)
        nv = (off + 15) // 16

        def sel_body(s, carry):
            selv, seli, pv, pi = carry

            def scan_body(k, c2):
                bestv, besti = c2
                sv = svv[pl.ds(k * 16, 16)]
                si = siv[pl.ds(k * 16, 16)]
                elig = (sv < pv) | ((sv == pv) & (si > pi))
                better = elig & ((sv > bestv) | ((sv == bestv) & (si < besti)))
                return (jnp.where(better, sv, bestv),
                        jnp.where(better, si, besti))

            bestv, besti = lax.fori_loop(
                0, nv, scan_body,
                (jnp.full((16,), NEGSEL, jnp.float32),
                 jnp.full((16,), IMAX, jnp.int32)))
            mv = bestv
            for sh in (8, 4, 2, 1):
                mv = jnp.maximum(mv, jnp.take(mv, lax.bitwise_xor(iota, sh)))
            mi = jnp.where(bestv == mv, besti, IMAX)
            for sh in (8, 4, 2, 1):
                mi = jnp.minimum(mi, jnp.take(mi, lax.bitwise_xor(iota, sh)))
            sel_here = iota == s
            return (jnp.where(sel_here, mv, selv),
                    jnp.where(sel_here, mi, seli),
                    mv, mi)

        selv, seli, _, _ = lax.fori_loop(
            0, TOPK, sel_body,
            (jnp.zeros((16,), jnp.float32), jnp.zeros((16,), jnp.int32),
             jnp.full((16,), 3.4e38, jnp.float32),
             jnp.full((16,), -1, jnp.int32)))
        jjv = seli // CH
        rem = seli - jjv * CH
        cidsel = plsc.load_gather(cid_v, [jjv + ql * 16])
        gsel = cidsel * CH + rem
        w = jnp.exp(selv)
        lbl = plsc.load_gather(by_v, [gsel])
        acc0 = jnp.zeros((16,), jnp.float32)
        acc1 = jnp.zeros((16,), jnp.float32)
        for sl in range(TOPK):
            l = lbl[sl]
            ws = w[sl]
            acc0 = acc0 + jnp.where(iota == l, ws, 0.0)
            acc1 = acc1 + jnp.where(iota == (l - 16), ws, 0.0)
        obuf[pl.ds(ql * NCLS, 16)] = acc0
        obuf[pl.ds(ql * NCLS + 16, 16)] = acc1

    _gather(0, gbufa, sema).start()
    _gather(1, gbufb, semb).start()

    def qbody(i, carry):
        ql0 = i * 2
        _gather(ql0, gbufa, sema).wait()
        _process(ql0, gbufa)

        @pl.when(ql0 + 2 < qpw)
        def _():
            _gather(ql0 + 2, gbufa, sema).start()

        _gather(ql0 + 1, gbufb, semb).wait()
        _process(ql0 + 1, gbufb)

        @pl.when(ql0 + 3 < qpw)
        def _():
            _gather(ql0 + 3, gbufb, semb).start()

        return carry

    lax.fori_loop(0, qpw // 2, qbody, 0)
    pltpu.sync_copy(obuf, out_hbm.at[pl.ds(q0 * NCLS, qpw * NCLS)])


def kernel(embedding_a, embedding_b, b_y):
    sim, mt = pl.pallas_call(
        _phase_a_body,
        grid=(NSTEP,),
        in_specs=[
            pl.BlockSpec((Q, D), lambda j: (0, 0)),
            pl.BlockSpec((KC, D), lambda j: (j, 0)),
        ],
        out_specs=[
            pl.BlockSpec((Q, KC), lambda j: (0, j)),
            pl.BlockSpec((KC // CH, Q), lambda j: (j, 0)),
        ],
        out_shape=[
            jax.ShapeDtypeStruct((Q, KP), jnp.float32),
            jax.ShapeDtypeStruct((NCHUNK, Q), jnp.float32),
        ],
    )(embedding_a, embedding_b)

    cidq = pl.pallas_call(
        _phase_b_body,
        out_shape=jax.ShapeDtypeStruct((Q, 16), jnp.int32),
    )(mt)

    cid_flat = cidq.reshape(Q * 16)
    sim_rows = sim.reshape(Q * NCHUNK, CH)

    sc = pl.kernel(
        _phase_c_body,
        out_type=jax.ShapeDtypeStruct((Q * NCLS,), jnp.float32),
        mesh=plsc.VectorSubcoreMesh(core_axis_name="c", subcore_axis_name="s"),
        compiler_params=pltpu.CompilerParams(needs_layout_passes=False),
        scratch_types=[
            pltpu.VMEM((K,), jnp.int32),
            pltpu.VMEM((Q // 32 * 16,), jnp.int32),
            pltpu.VMEM((16, CH), jnp.float32),
            pltpu.VMEM((16, CH), jnp.float32),
            pltpu.VMEM((TOPK * CH + 32,), jnp.float32),
            pltpu.VMEM((TOPK * CH + 32,), jnp.int32),
            pltpu.VMEM((Q // 32 * NCLS,), jnp.float32),
            pltpu.SemaphoreType.DMA,
            pltpu.SemaphoreType.DMA,
        ],
    )
    return sc(sim_rows, cid_flat, b_y).reshape(Q, NCLS)


# speedup vs baseline: 9.4277x; 1.9917x over previous
import functools

import jax
import jax.numpy as jnp
from jax import lax
from jax.experimental import pallas as pl
from jax.experimental.pallas import tpu as pltpu
from jax.experimental.pallas import tpu_sc as plsc

Q = 1024
D = 64
K = 100000
CH = 128
KC = 2048
KP = 100352
NCHUNK = KP // CH
NSTEP = KP // KC
NCLS = 32
TOPK = 10
EPS = 1e-8
NEG = -3.0
NEGSEL = -3.4e38
IMAX = 2**31 - 1


def _phase_a_body(a_ref, b_ref, sim_ref, mt_ref):
    j = pl.program_id(0)
    a = a_ref[...]
    an = jnp.sqrt(jnp.sum(a * a, axis=1, keepdims=True))
    a = a / jnp.maximum(an, EPS)
    b = b_ref[...]
    bn = jnp.sqrt(jnp.sum(b * b, axis=1, keepdims=True))
    b = b / jnp.maximum(bn, EPS)
    sim = lax.dot_general(a, b, (((1,), (1,)), ((), ())),
                          preferred_element_type=jnp.float32)
    kcol = j * KC + lax.broadcasted_iota(jnp.int32, (Q, KC), 1)
    sim = jnp.where(kcol < K, sim, NEG)
    rows = []
    for t in range(KC // CH):
        sub = sim[:, t * CH:(t + 1) * CH]
        sim_ref[pl.ds(t, 1), :, :] = sub[None]
        rows.append(jnp.max(sub, axis=1)[None, :])
    mt_ref[...] = jnp.concatenate(rows, axis=0)


def _phase_b_body(mt_ref, cid_ref):
    work = mt_ref[...]
    row_iota = lax.broadcasted_iota(jnp.int32, (NCHUNK, Q), 0)
    mv = None
    cids = []
    for s in range(TOPK):
        mv = jnp.max(work, axis=0)
        cid = jnp.min(jnp.where(work == mv[None, :], row_iota, IMAX), axis=0)
        cids.append(cid)
        work = jnp.where(row_iota == cid[None, :], -jnp.inf, work)
    for a in range(TOPK - 1):
        for b in range(TOPK - 1 - a):
            lo = jnp.minimum(cids[b], cids[b + 1])
            hi = jnp.maximum(cids[b], cids[b + 1])
            cids[b], cids[b + 1] = lo, hi
    cols = [c[:, None] for c in cids]
    cols.append(lax.bitcast_convert_type(mv, jnp.int32)[:, None])
    cols.append(jnp.zeros((Q, 16 - TOPK - 1), jnp.int32))
    cid_ref[...] = jnp.concatenate(cols, axis=1)


def _phase_c_body(sim_hbm, cid_hbm, by_hbm, out_hbm,
                  by_v, cid_v, gbufa, gbufb, svv, siv, obuf,
                  sema, semb):
    nc = 2
    wid = lax.axis_index("s") * nc + lax.axis_index("c")
    qpw = Q // 32
    q0 = wid * qpw
    pltpu.sync_copy(by_hbm, by_v)
    pltpu.sync_copy(cid_hbm.at[pl.ds(q0 * 16, qpw * 16)], cid_v)
    iota = lax.iota(jnp.int32, 16)

    def _gather(ql, buf, sem):
        cidvec = cid_v[pl.ds(ql * 16, 16)]
        gidx = jnp.where(iota < TOPK, cidvec, 0) * Q + (q0 + ql)
        return pltpu.make_async_copy(sim_hbm.at[gidx], buf, sem)

    def _process(ql, buf):
        cidvec = cid_v[pl.ds(ql * 16, 16)]
        thrs = plsc.bitcast(cidvec, jnp.float32)[TOPK]
        off = jnp.int32(0)
        for jj in range(TOPK):
            for t in range(CH // 16):
                v = buf[jj, pl.ds(t * 16, 16)]
                keep = v >= thrs
                p = iota + (jj * CH + t * 16)
                plsc.store_compressed(svv.at[pl.ds(off, 16)], v, mask=keep)
                plsc.store_compressed(siv.at[pl.ds(off, 16)], p, mask=keep)
                cnt = plsc.all_reduce_population_count(keep)
                cnt = cnt[0] if jnp.ndim(cnt) else cnt
                off = off + cnt
        svv[pl.ds(off, 16)] = jnp.full((16,), NEGSEL, jnp.float32)
        siv[pl.ds(off, 16)] = jnp.full((16,), IMAX, jnp.int32)
        nv = (off + 15) // 16

        def sel_body(s, carry):
            selv, seli, pv, pi = carry

            def scan_body(k, c2):
                bestv, besti = c2
                sv = svv[pl.ds(k * 16, 16)]
                si = siv[pl.ds(k * 16, 16)]
                elig = (sv < pv) | ((sv == pv) & (si > pi))
                better = elig & ((sv > bestv) | ((sv == bestv) & (si < besti)))
                return (jnp.where(better, sv, bestv),
                        jnp.where(better, si, besti))

            bestv, besti = lax.fori_loop(
                0, nv, scan_body,
                (jnp.full((16,), NEGSEL, jnp.float32),
                 jnp.full((16,), IMAX, jnp.int32)))
            mv = bestv
            for sh in (8, 4, 2, 1):
                mv = jnp.maximum(mv, jnp.take(mv, lax.bitwise_xor(iota, sh)))
            mi = jnp.where(bestv == mv, besti, IMAX)
            for sh in (8, 4, 2, 1):
                mi = jnp.minimum(mi, jnp.take(mi, lax.bitwise_xor(iota, sh)))
            sel_here = iota == s
            return (jnp.where(sel_here, mv, selv),
                    jnp.where(sel_here, mi, seli),
                    mv, mi)

        selv, seli, _, _ = lax.fori_loop(
            0, TOPK, sel_body,
            (jnp.zeros((16,), jnp.float32), jnp.zeros((16,), jnp.int32),
             jnp.full((16,), 3.4e38, jnp.float32),
             jnp.full((16,), -1, jnp.int32)))
        jjv = seli // CH
        rem = seli - jjv * CH
        cidsel = plsc.load_gather(cid_v, [jjv + ql * 16])
        gsel = cidsel * CH + rem
        w = jnp.exp(selv)
        lbl = plsc.load_gather(by_v, [gsel])
        acc0 = jnp.zeros((16,), jnp.float32)
        acc1 = jnp.zeros((16,), jnp.float32)
        for sl in range(TOPK):
            l = lbl[sl]
            ws = w[sl]
            acc0 = acc0 + jnp.where(iota == l, ws, 0.0)
            acc1 = acc1 + jnp.where(iota == (l - 16), ws, 0.0)
        obuf[pl.ds(ql * NCLS, 16)] = acc0
        obuf[pl.ds(ql * NCLS + 16, 16)] = acc1

    _gather(0, gbufa, sema).start()
    _gather(1, gbufb, semb).start()

    def qbody(i, carry):
        ql0 = i * 2
        _gather(ql0, gbufa, sema).wait()
        _process(ql0, gbufa)

        @pl.when(ql0 + 2 < qpw)
        def _():
            _gather(ql0 + 2, gbufa, sema).start()

        _gather(ql0 + 1, gbufb, semb).wait()
        _process(ql0 + 1, gbufb)

        @pl.when(ql0 + 3 < qpw)
        def _():
            _gather(ql0 + 3, gbufb, semb).start()

        return carry

    lax.fori_loop(0, qpw // 2, qbody, 0)
    pltpu.sync_copy(obuf, out_hbm.at[pl.ds(q0 * NCLS, qpw * NCLS)])


def kernel(embedding_a, embedding_b, b_y):
    sim, mt = pl.pallas_call(
        _phase_a_body,
        grid=(NSTEP,),
        in_specs=[
            pl.BlockSpec((Q, D), lambda j: (0, 0)),
            pl.BlockSpec((KC, D), lambda j: (j, 0)),
        ],
        out_specs=[
            pl.BlockSpec((KC // CH, Q, CH), lambda j: (j, 0, 0)),
            pl.BlockSpec((KC // CH, Q), lambda j: (j, 0)),
        ],
        out_shape=[
            jax.ShapeDtypeStruct((NCHUNK, Q, CH), jnp.float32),
            jax.ShapeDtypeStruct((NCHUNK, Q), jnp.float32),
        ],
    )(embedding_a, embedding_b)

    cidq = pl.pallas_call(
        _phase_b_body,
        out_shape=jax.ShapeDtypeStruct((Q, 16), jnp.int32),
    )(mt)

    cid_flat = cidq.reshape(Q * 16)
    sim_rows = sim.reshape(NCHUNK * Q, CH)

    sc = pl.kernel(
        _phase_c_body,
        out_type=jax.ShapeDtypeStruct((Q * NCLS,), jnp.float32),
        mesh=plsc.VectorSubcoreMesh(core_axis_name="c", subcore_axis_name="s"),
        compiler_params=pltpu.CompilerParams(needs_layout_passes=False),
        scratch_types=[
            pltpu.VMEM((K,), jnp.int32),
            pltpu.VMEM((Q // 32 * 16,), jnp.int32),
            pltpu.VMEM((16, CH), jnp.float32),
            pltpu.VMEM((16, CH), jnp.float32),
            pltpu.VMEM((TOPK * CH + 32,), jnp.float32),
            pltpu.VMEM((TOPK * CH + 32,), jnp.int32),
            pltpu.VMEM((Q // 32 * NCLS,), jnp.float32),
            pltpu.SemaphoreType.DMA,
            pltpu.SemaphoreType.DMA,
        ],
    )
    return sc(sim_rows, cid_flat, b_y).reshape(Q, NCLS)


# speedup vs baseline: 12.2354x; 1.2978x over previous
import functools

import jax
import jax.numpy as jnp
from jax import lax
from jax.experimental import pallas as pl
from jax.experimental.pallas import tpu as pltpu
from jax.experimental.pallas import tpu_sc as plsc

Q = 1024
D = 64
K = 100000
CH = 128
KC = 2048
KP = 100352
NCHUNK = KP // CH
NSTEP = KP // KC
NCLS = 32
TOPK = 10
EPS = 1e-8
NEG = -3.0
NEGSEL = -3.4e38
IMAX = 2**31 - 1


def _phase_a_body(a_ref, b_ref, sim_ref, mt_ref):
    j = pl.program_id(0)
    a = a_ref[...]
    an = jnp.sqrt(jnp.sum(a * a, axis=1, keepdims=True))
    a = a / jnp.maximum(an, EPS)
    b = b_ref[...]
    bn = jnp.sqrt(jnp.sum(b * b, axis=0, keepdims=True))
    b = b / jnp.maximum(bn, EPS)
    sim = lax.dot_general(a, b, (((1,), (0,)), ((), ())),
                          preferred_element_type=jnp.float32)
    kcol = j * KC + lax.broadcasted_iota(jnp.int32, (Q, KC), 1)
    sim = jnp.where(kcol < K, sim, NEG)
    rows = []
    for t in range(KC // CH):
        sub = sim[:, t * CH:(t + 1) * CH]
        sim_ref[pl.ds(t, 1), :, :] = sub[None]
        rows.append(jnp.max(sub, axis=1)[None, :])
    mt_ref[...] = jnp.concatenate(rows, axis=0)


def _phase_b_body(mt_ref, cid_ref):
    work = mt_ref[...]
    row_iota = lax.broadcasted_iota(jnp.int32, (NCHUNK, Q), 0)
    mv = None
    cids = []
    for s in range(TOPK):
        mv = jnp.max(work, axis=0)
        cid = jnp.min(jnp.where(work == mv[None, :], row_iota, IMAX), axis=0)
        cids.append(cid)
        work = jnp.where(row_iota == cid[None, :], -jnp.inf, work)
    for a in range(TOPK - 1):
        for b in range(TOPK - 1 - a):
            lo = jnp.minimum(cids[b], cids[b + 1])
            hi = jnp.maximum(cids[b], cids[b + 1])
            cids[b], cids[b + 1] = lo, hi
    cols = [c[:, None] for c in cids]
    cols.append(lax.bitcast_convert_type(mv, jnp.int32)[:, None])
    cols.append(jnp.zeros((Q, 16 - TOPK - 1), jnp.int32))
    cid_ref[...] = jnp.concatenate(cols, axis=1)


def _phase_c_body(sim_hbm, cid_hbm, by_hbm, out_hbm,
                  by_v, cid_v, gbufa, gbufb, svv, siv, obuf,
                  sema, semb):
    nc = 2
    wid = lax.axis_index("s") * nc + lax.axis_index("c")
    qpw = Q // 32
    q0 = wid * qpw
    pltpu.sync_copy(by_hbm, by_v)
    pltpu.sync_copy(cid_hbm.at[pl.ds(q0 * 16, qpw * 16)], cid_v)
    iota = lax.iota(jnp.int32, 16)

    def _gather(ql, buf, sem):
        cidvec = cid_v[pl.ds(ql * 16, 16)]
        gidx = jnp.where(iota < TOPK, cidvec, 0) * Q + (q0 + ql)
        return pltpu.make_async_copy(sim_hbm.at[gidx], buf, sem)

    def _process(ql, buf):
        cidvec = cid_v[pl.ds(ql * 16, 16)]
        thrs = plsc.bitcast(cidvec, jnp.float32)[TOPK]
        off = jnp.int32(0)
        for jj in range(TOPK):
            for t in range(CH // 16):
                v = buf[jj, pl.ds(t * 16, 16)]
                keep = v >= thrs
                p = iota + (jj * CH + t * 16)
                plsc.store_compressed(svv.at[pl.ds(off, 16)], v, mask=keep)
                plsc.store_compressed(siv.at[pl.ds(off, 16)], p, mask=keep)
                cnt = plsc.all_reduce_population_count(keep)
                cnt = cnt[0] if jnp.ndim(cnt) else cnt
                off = off + cnt
        svv[pl.ds(off, 16)] = jnp.full((16,), NEGSEL, jnp.float32)
        siv[pl.ds(off, 16)] = jnp.zeros((16,), jnp.int32)
        nv = (off + 15) // 16

        def sel_body(s, carry):
            selv, seli, pv, pi = carry

            def scan_body(k, c2):
                bestv, besti = c2
                sv = svv[pl.ds(k * 16, 16)]
                si = siv[pl.ds(k * 16, 16)]
                elig = (sv < pv) | ((sv == pv) & (si > pi))
                better = elig & ((sv > bestv) | ((sv == bestv) & (si < besti)))
                return (jnp.where(better, sv, bestv),
                        jnp.where(better, si, besti))

            bestv, besti = lax.fori_loop(
                0, nv, scan_body,
                (jnp.full((16,), NEGSEL, jnp.float32),
                 jnp.full((16,), IMAX, jnp.int32)))
            mv = bestv
            for sh in (8, 4, 2, 1):
                mv = jnp.maximum(mv, jnp.take(mv, lax.bitwise_xor(iota, sh)))
            mi = jnp.where(bestv == mv, besti, IMAX)
            for sh in (8, 4, 2, 1):
                mi = jnp.minimum(mi, jnp.take(mi, lax.bitwise_xor(iota, sh)))
            sel_here = iota == s
            return (jnp.where(sel_here, mv, selv),
                    jnp.where(sel_here, mi, seli),
                    mv, mi)

        def slow_path():
            selv, seli, _, _ = lax.fori_loop(
                0, TOPK, sel_body,
                (jnp.zeros((16,), jnp.float32), jnp.zeros((16,), jnp.int32),
                 jnp.full((16,), 3.4e38, jnp.float32),
                 jnp.full((16,), -1, jnp.int32)))
            return selv, seli

        def fast_path():
            return svv[pl.ds(0, 16)], siv[pl.ds(0, 16)]

        selv, seli = lax.cond(off == TOPK, fast_path, slow_path)
        jjv = seli // CH
        rem = seli - jjv * CH
        cidsel = plsc.load_gather(cid_v, [jjv + ql * 16])
        gsel = cidsel * CH + rem
        w = jnp.exp(selv)
        lbl = plsc.load_gather(by_v, [gsel])
        acc0 = jnp.zeros((16,), jnp.float32)
        acc1 = jnp.zeros((16,), jnp.float32)
        for sl in range(TOPK):
            l = lbl[sl]
            ws = w[sl]
            acc0 = acc0 + jnp.where(iota == l, ws, 0.0)
            acc1 = acc1 + jnp.where(iota == (l - 16), ws, 0.0)
        obuf[pl.ds(ql * NCLS, 16)] = acc0
        obuf[pl.ds(ql * NCLS + 16, 16)] = acc1

    _gather(0, gbufa, sema).start()
    _gather(1, gbufb, semb).start()

    def qbody(i, carry):
        ql0 = i * 2
        _gather(ql0, gbufa, sema).wait()
        _process(ql0, gbufa)

        @pl.when(ql0 + 2 < qpw)
        def _():
            _gather(ql0 + 2, gbufa, sema).start()

        _gather(ql0 + 1, gbufb, semb).wait()
        _process(ql0 + 1, gbufb)

        @pl.when(ql0 + 3 < qpw)
        def _():
            _gather(ql0 + 3, gbufb, semb).start()

        return carry

    lax.fori_loop(0, qpw // 2, qbody, 0)
    pltpu.sync_copy(obuf, out_hbm.at[pl.ds(q0 * NCLS, qpw * NCLS)])


def kernel(embedding_a, embedding_b, b_y):
    sim, mt = pl.pallas_call(
        _phase_a_body,
        grid=(NSTEP,),
        in_specs=[
            pl.BlockSpec((Q, D), lambda j: (0, 0)),
            pl.BlockSpec((D, KC), lambda j: (0, j)),
        ],
        out_specs=[
            pl.BlockSpec((KC // CH, Q, CH), lambda j: (j, 0, 0)),
            pl.BlockSpec((KC // CH, Q), lambda j: (j, 0)),
        ],
        out_shape=[
            jax.ShapeDtypeStruct((NCHUNK, Q, CH), jnp.float32),
            jax.ShapeDtypeStruct((NCHUNK, Q), jnp.float32),
        ],
    )(embedding_a, embedding_b.T)

    cidq = pl.pallas_call(
        _phase_b_body,
        out_shape=jax.ShapeDtypeStruct((Q, 16), jnp.int32),
    )(mt)

    cid_flat = cidq.reshape(Q * 16)
    sim_rows = sim.reshape(NCHUNK * Q, CH)

    sc = pl.kernel(
        _phase_c_body,
        out_type=jax.ShapeDtypeStruct((Q * NCLS,), jnp.float32),
        mesh=plsc.VectorSubcoreMesh(core_axis_name="c", subcore_axis_name="s"),
        compiler_params=pltpu.CompilerParams(needs_layout_passes=False),
        scratch_types=[
            pltpu.VMEM((K,), jnp.int32),
            pltpu.VMEM((Q // 32 * 16,), jnp.int32),
            pltpu.VMEM((16, CH), jnp.float32),
            pltpu.VMEM((16, CH), jnp.float32),
            pltpu.VMEM((TOPK * CH + 32,), jnp.float32),
            pltpu.VMEM((TOPK * CH + 32,), jnp.int32),
            pltpu.VMEM((Q // 32 * NCLS,), jnp.float32),
            pltpu.SemaphoreType.DMA,
            pltpu.SemaphoreType.DMA,
        ],
    )
    return sc(sim_rows, cid_flat, b_y).reshape(Q, NCLS)
